# Initial kernel scaffold; baseline (speedup 1.0000x reference)
import jax
import jax.numpy as jnp
from jax.experimental import pallas as pl


def kernel(x, Wq, bq, Wk, bk, Wv, bv, Wp, bp, Wg, bg):
    raise NotImplementedError("write your pallas kernel here")


# speedup vs baseline: 2.4923x; 2.4923x over previous
import functools

import jax
import jax.numpy as jnp
from jax import lax
from jax.experimental import pallas as pl
from jax.experimental.pallas import tpu as pltpu
from jax.experimental.pallas import tpu_sc as plsc

WS = 8
GS = 9
C = 96
CP = 128
WS2 = WS * WS
GS2 = GS * GS
N = GS2 * WS2
CS = WS2
KW = 2 * CS
PAD = CS // 2
NPAD = N + 2 * PAD
SCALE = C ** (-0.5)

VR = N // 16
HB = 96
GCH = 96
NCHUNK = N // GCH


def _ids_body(xf_ref, ids_ref):
    xs = xf_ref[0]
    means = xs.reshape(GS2, WS2, CP).sum(axis=1) * (1.0 / WS2)
    simT = lax.dot_general(means, xs, (((1,), (1,)), ((), ())),
                           preferred_element_type=jnp.float32)
    mx = jnp.max(simT, axis=0, keepdims=True)
    row = lax.broadcasted_iota(jnp.int32, simT.shape, 0)
    cand = jnp.where(simT == mx, row, jnp.int32(1 << 20))
    ids_ref[0] = jnp.min(cand, axis=0, keepdims=True)


def _attn_body(xs_ref, idc_ref, idr_ref, id2_ref, wq, bq, wk, bk, wv, bv,
               wp, bp, wg, bg, out_ref, kp, vp, qb, gb, ob):
    nt = (((1,), (1,)), ((), ()))
    xs = xs_ref[0]
    qb[...] = lax.dot_general(xs, wq[...], nt,
                              preferred_element_type=jnp.float32) + bq[...]
    zpad = jnp.zeros((PAD, CP), jnp.float32)
    kp[pl.ds(0, PAD), :] = zpad
    kp[pl.ds(PAD + N, PAD), :] = zpad
    kp[pl.ds(PAD, N), :] = lax.dot_general(xs, wk[...], nt,
                                           preferred_element_type=jnp.float32) + bk[...]
    vp[pl.ds(0, PAD), :] = zpad
    vp[pl.ds(PAD + N, PAD), :] = zpad
    vp[pl.ds(PAD, N), :] = lax.dot_general(xs, wv[...], nt,
                                           preferred_element_type=jnp.float32) + bv[...]
    gb[...] = jax.nn.sigmoid(
        lax.dot_general(xs, wg[...], nt, preferred_element_type=jnp.float32)
        + bg[...])

    def chunk(n, _):
        q_c = qb[pl.ds(n * CS, CS), :]
        kw_c = kp[pl.ds(n * CS, KW), :]
        vw_c = vp[pl.ds(n * CS, KW), :]
        qid = idc_ref[0, pl.ds(n * CS, CS), :]
        half = (n // 2) * KW
        kid_e = idr_ref[0, :, pl.ds(half, KW)]
        kid_o = id2_ref[0, :, pl.ds(half, KW)]
        kid = jnp.where(n % 2 == 0, kid_e, kid_o)
        s = lax.dot_general(q_c, kw_c, (((1,), (1,)), ((), ())),
                            preferred_element_type=jnp.float32) * SCALE
        s = jnp.where(qid == kid, s, jnp.float32(-10000.0))
        s = s - jnp.max(s, axis=1, keepdims=True)
        p = jnp.exp(s)
        p = p / jnp.sum(p, axis=1, keepdims=True)
        o = lax.dot_general(p, vw_c, (((1,), (0,)), ((), ())),
                            preferred_element_type=jnp.float32)
        ob[pl.ds(n * CS, CS), :] = o * gb[pl.ds(n * CS, CS), :]
        return 0

    lax.fori_loop(0, GS2, chunk, 0)
    out_ref[0] = lax.dot_general(ob[...], wp[...], nt,
                                 preferred_element_type=jnp.float32) + bp[...]


_TAKE_DN = lax.GatherDimensionNumbers(
    offset_dims=(), collapsed_slice_dims=(0,), start_index_map=(0,))


def _take16(v, idx):
    return lax.gather(v, idx[:, None], _TAKE_DN, (1,),
                      mode=lax.GatherScatterMode.PROMISE_IN_BOUNDS)


def _sort_gather_body(ids_hbm, xf_hbm, xs_hbm, idsrt_hbm, pos_hbm,
                      ids_v, r_v, pos_v, sidx_v, idp_v, hist_v, fx_v,
                      gbuf, sem):
    wid = lax.axis_index("s") * 2 + lax.axis_index("c")
    base = wid * N
    pltpu.sync_copy(ids_hbm.at[wid], ids_v)
    iota = lax.iota(jnp.int32, 16)
    zero16 = jnp.zeros((16,), jnp.int32)
    one16 = jnp.ones((16,), jnp.int32)
    neg16 = jnp.full((16,), -1, jnp.int32)
    for t in range(HB // 16):
        hist_v[pl.ds(t * 16, 16)] = zero16
    idp_v[pl.ds(0, 16)] = neg16
    idp_v[pl.ds(16, 16)] = neg16
    idp_v[pl.ds(PAD + N, 16)] = neg16
    idp_v[pl.ds(PAD + N + 16, 16)] = neg16

    def pass_a(j, _):
        idv = ids_v[pl.ds(j * 16, 16)]
        w = zero16
        e = zero16
        for s in range(1, 16):
            shf = _take16(idv, jnp.maximum(iota - s, 0))
            w = w + jnp.where((iota >= s) & (idv == shf), one16, zero16)
            shb = _take16(idv, jnp.minimum(iota + s, 15))
            e = e + jnp.where((iota < 16 - s) & (idv == shb), one16, zero16)
        cnt = plsc.load_gather(hist_v, [idv])
        r_v[pl.ds(j * 16, 16)] = cnt + w
        plsc.store_scatter(hist_v, [idv], cnt + w + e + 1, mask=(e == 0))
        return 0

    lax.fori_loop(0, VR, pass_a, 0)

    carry = jnp.int32(0)
    for t in range(HB // 16):
        h = hist_v[pl.ds(t * 16, 16)]
        inc = plsc.cumsum(h) + carry
        fx_v[pl.ds(t * 16, 16)] = inc - h
        carry = jnp.max(inc)

    def pass_b(j, _):
        idv = ids_v[pl.ds(j * 16, 16)]
        p = plsc.load_gather(fx_v, [idv]) + r_v[pl.ds(j * 16, 16)]
        pos_v[pl.ds(j * 16, 16)] = p + base
        plsc.store_scatter(sidx_v, [p], iota + (base + j * 16))
        plsc.store_scatter(idp_v, [p + PAD], idv)
        return 0

    lax.fori_loop(0, VR, pass_b, 0)

    pltpu.sync_copy(pos_v, pos_hbm.at[wid])
    pltpu.sync_copy(idp_v, idsrt_hbm.at[wid])
    for cc in range(NCHUNK):
        pltpu.async_copy(xf_hbm.at[sidx_v.at[pl.ds(cc * GCH, GCH)]],
                         gbuf, sem).wait()
        pltpu.sync_copy(gbuf, xs_hbm.at[pl.ds(base + cc * GCH, GCH)])


def _unsort_body(pos_hbm, os_hbm, out_hbm, pos_v, gbuf, sem):
    wid = lax.axis_index("s") * 2 + lax.axis_index("c")
    base = wid * N
    pltpu.sync_copy(pos_hbm.at[wid], pos_v)
    for cc in range(NCHUNK):
        pltpu.async_copy(os_hbm.at[pos_v.at[pl.ds(cc * GCH, GCH)]],
                         gbuf, sem).wait()
        pltpu.sync_copy(gbuf, out_hbm.at[pl.ds(base + cc * GCH, GCH)])


def _make_calls(BN):
    ids_call = pl.pallas_call(
        _ids_body,
        grid=(BN,),
        in_specs=[pl.BlockSpec((1, N, CP), lambda i: (i, 0, 0))],
        out_specs=pl.BlockSpec((1, 1, N), lambda i: (i, 0, 0)),
        out_shape=jax.ShapeDtypeStruct((BN, 1, N), jnp.int32),
    )

    wspec = pl.BlockSpec((CP, CP), lambda i: (0, 0))
    bspec = pl.BlockSpec((1, CP), lambda i: (0, 0))
    attn_call = pl.pallas_call(
        _attn_body,
        grid=(BN,),
        in_specs=[pl.BlockSpec((1, N, CP), lambda i: (i, 0, 0)),
                  pl.BlockSpec((1, N, 1), lambda i: (i, 0, 0)),
                  pl.BlockSpec((1, 1, NPAD), lambda i: (i, 0, 0)),
                  pl.BlockSpec((1, 1, NPAD), lambda i: (i, 0, 0)),
                  wspec, bspec, wspec, bspec, wspec, bspec, wspec, bspec,
                  wspec, bspec],
        out_specs=pl.BlockSpec((1, N, CP), lambda i: (i, 0, 0)),
        out_shape=jax.ShapeDtypeStruct((BN, N, CP), jnp.float32),
        scratch_shapes=[pltpu.VMEM((NPAD, CP), jnp.float32),
                        pltpu.VMEM((NPAD, CP), jnp.float32),
                        pltpu.VMEM((N, CP), jnp.float32),
                        pltpu.VMEM((N, CP), jnp.float32),
                        pltpu.VMEM((N, CP), jnp.float32)],
    )

    mesh = plsc.VectorSubcoreMesh(core_axis_name="c", subcore_axis_name="s")
    sort_call = functools.partial(
        pl.kernel,
        out_type=[jax.ShapeDtypeStruct((BN * N, CP), jnp.float32),
                  jax.ShapeDtypeStruct((BN, NPAD), jnp.int32),
                  jax.ShapeDtypeStruct((BN, N), jnp.int32)],
        mesh=mesh,
        compiler_params=pltpu.CompilerParams(needs_layout_passes=False),
        scratch_types=[pltpu.VMEM((N,), jnp.int32),
                       pltpu.VMEM((N,), jnp.int32),
                       pltpu.VMEM((N,), jnp.int32),
                       pltpu.VMEM((N,), jnp.int32),
                       pltpu.VMEM((NPAD,), jnp.int32),
                       pltpu.VMEM((HB,), jnp.int32),
                       pltpu.VMEM((HB,), jnp.int32),
                       pltpu.VMEM((GCH, CP), jnp.float32),
                       pltpu.SemaphoreType.DMA],
    )(_sort_gather_body)

    unsort_call = functools.partial(
        pl.kernel,
        out_type=jax.ShapeDtypeStruct((BN * N, CP), jnp.float32),
        mesh=mesh,
        compiler_params=pltpu.CompilerParams(needs_layout_passes=False),
        scratch_types=[pltpu.VMEM((N,), jnp.int32),
                       pltpu.VMEM((GCH, CP), jnp.float32),
                       pltpu.SemaphoreType.DMA],
    )(_unsort_body)

    return ids_call, sort_call, attn_call, unsort_call


def kernel(x, Wq, bq, Wk, bk, Wv, bv, Wp, bp, Wg, bg):
    B, Cc, H, W = x.shape
    gh, gw = H // (WS * GS), W // (WS * GS)
    ng = gh * gw
    BN = B * ng

    xg = x.reshape(B, Cc, gh, GS, WS, gw, GS, WS)
    xg = jnp.transpose(xg, (0, 2, 5, 3, 6, 4, 7, 1))
    xf = xg.reshape(BN, N, Cc)
    xf = jnp.pad(xf, ((0, 0), (0, 0), (0, CP - Cc)))

    ids_call, sort_call, attn_call, unsort_call = _make_calls(BN)

    ids3 = ids_call(xf)
    ids2 = ids3.reshape(BN, N)
    xs_flat, idsrt, pos = sort_call(ids2, xf.reshape(BN * N, CP))
    idc = idsrt[:, PAD:PAD + N].reshape(BN, N, 1)
    idr = idsrt.reshape(BN, 1, NPAD)
    idr2 = jnp.pad(idsrt[:, KW // 2:], ((0, 0), (0, KW // 2)),
                   constant_values=-1).reshape(BN, 1, NPAD)
    wpad = lambda w: jnp.pad(w, ((0, CP - Cc), (0, CP - Cc)))
    bpad = lambda b: jnp.pad(b, (0, CP - Cc)).reshape(1, CP)
    osort = attn_call(xs_flat.reshape(BN, N, CP), idc, idr, idr2,
                      wpad(Wq), bpad(bq), wpad(Wk), bpad(bk),
                      wpad(Wv), bpad(bv), wpad(Wp), bpad(bp),
                      wpad(Wg), bpad(bg))
    ofin = unsort_call(pos, osort.reshape(BN * N, CP))

    out = ofin.reshape(B, gh, gw, GS, GS, WS, WS, CP)
    out = jnp.transpose(out, (0, 7, 1, 3, 5, 2, 4, 6))
    return out[:, :Cc].reshape(B, Cc, H, W)


# speedup vs baseline: 4.3237x; 1.7348x over previous
import functools

import jax
import jax.numpy as jnp
from jax import lax
from jax.experimental import pallas as pl
from jax.experimental.pallas import tpu as pltpu
from jax.experimental.pallas import tpu_sc as plsc

WS = 8
GS = 9
C = 96
CP = 128
WS2 = WS * WS
GS2 = GS * GS
N = GS2 * WS2
CS = WS2
KW = 2 * CS
BQ = 3 * CS
KWB = BQ + CS
NB = GS2 // 3
PAD = CS // 2
NPAD = N + 2 * PAD
SCALE = C ** (-0.5)

VR = N // 16
HB = 96
GCH = 96
NCHUNK = N // GCH


def _ids_body(xf_ref, ids_ref):
    xs = xf_ref[0]
    means = xs.reshape(GS2, WS2, CP).sum(axis=1) * (1.0 / WS2)
    simT = lax.dot_general(means, xs, (((1,), (1,)), ((), ())),
                           preferred_element_type=jnp.float32)
    mx = jnp.max(simT, axis=0, keepdims=True)
    row = lax.broadcasted_iota(jnp.int32, simT.shape, 0)
    cand = jnp.where(simT == mx, row, jnp.int32(1 << 20))
    ids_ref[0] = jnp.min(cand, axis=0, keepdims=True)


def _attn_body(xs_ref, idc_ref, idr_ref, id2_ref, wq, bq, wk, bk, wv, bv,
               wp, bp, wg, bg, out_ref, kp, vp, qb, gb, ob):
    nt = (((1,), (1,)), ((), ()))
    xs = xs_ref[0]
    qb[...] = lax.dot_general(xs, wq[...], nt,
                              preferred_element_type=jnp.float32) + bq[...]
    zpad = jnp.zeros((PAD, CP), jnp.float32)
    kp[pl.ds(0, PAD), :] = zpad
    kp[pl.ds(PAD + N, PAD), :] = zpad
    kp[pl.ds(PAD, N), :] = lax.dot_general(xs, wk[...], nt,
                                           preferred_element_type=jnp.float32) + bk[...]
    vp[pl.ds(0, PAD), :] = zpad
    vp[pl.ds(PAD + N, PAD), :] = zpad
    vp[pl.ds(PAD, N), :] = lax.dot_general(xs, wv[...], nt,
                                           preferred_element_type=jnp.float32) + bv[...]
    gb[...] = jax.nn.sigmoid(
        lax.dot_general(xs, wg[...], nt, preferred_element_type=jnp.float32)
        + bg[...])

    row = lax.broadcasted_iota(jnp.int32, (BQ, KWB), 0)
    col = lax.broadcasted_iota(jnp.int32, (BQ, KWB), 1)
    wstart = (row // CS) * CS
    inwin = (col >= wstart) & (col < wstart + KW)

    def block(nb):
        q_c = qb[pl.ds(nb * BQ, BQ), :]
        kw_c = kp[pl.ds(nb * BQ, KWB), :]
        vw_c = vp[pl.ds(nb * BQ, KWB), :]
        qid = idc_ref[0, pl.ds(nb * BQ, BQ), :]
        off = ((nb * 3) // 2) * 128
        kid_e = idr_ref[0, :, pl.ds(off, KWB)]
        kid_o = id2_ref[0, :, pl.ds(off, KWB)]
        kid = jnp.where(nb % 2 == 0, kid_e, kid_o)
        s = lax.dot_general(q_c, kw_c, (((1,), (1,)), ((), ())),
                            preferred_element_type=jnp.float32) * SCALE
        s = jnp.where((qid == kid) & inwin, s, jnp.float32(-10000.0))
        p = jnp.exp(s - jnp.max(s, axis=1, keepdims=True))
        den = jnp.sum(p, axis=1, keepdims=True)
        o = lax.dot_general(p, vw_c, (((1,), (0,)), ((), ())),
                            preferred_element_type=jnp.float32)
        ob[pl.ds(nb * BQ, BQ), :] = (o / den) * gb[pl.ds(nb * BQ, BQ), :]

    def iter3(i, _):
        for u in range(3):
            block(i * 3 + u)
        return 0

    lax.fori_loop(0, NB // 3, iter3, 0)
    out_ref[0] = lax.dot_general(ob[...], wp[...], nt,
                                 preferred_element_type=jnp.float32) + bp[...]


_TAKE_DN = lax.GatherDimensionNumbers(
    offset_dims=(), collapsed_slice_dims=(0,), start_index_map=(0,))


def _take16(v, idx):
    return lax.gather(v, idx[:, None], _TAKE_DN, (1,),
                      mode=lax.GatherScatterMode.PROMISE_IN_BOUNDS)


def _sort_gather_body(nslice, ids_hbm, xf_hbm, xs_hbm, idsrt_hbm, pos_hbm,
                      ids_v, r_v, pos_v, sidx_v, idp_v, hist_v, fx_v,
                      gbuf, sem):
    wid = lax.axis_index("s") * 2 + lax.axis_index("c")
    @pl.when(wid < nslice)
    def _():
        _sort_gather_work(wid, ids_hbm, xf_hbm, xs_hbm, idsrt_hbm, pos_hbm,
                          ids_v, r_v, pos_v, sidx_v, idp_v, hist_v, fx_v,
                          gbuf, sem)


def _sort_gather_work(wid, ids_hbm, xf_hbm, xs_hbm, idsrt_hbm, pos_hbm,
                      ids_v, r_v, pos_v, sidx_v, idp_v, hist_v, fx_v,
                      gbuf, sem):
    base = wid * N
    pltpu.sync_copy(ids_hbm.at[wid, 0], ids_v)
    iota = lax.iota(jnp.int32, 16)
    zero16 = jnp.zeros((16,), jnp.int32)
    one16 = jnp.ones((16,), jnp.int32)
    neg16 = jnp.full((16,), -1, jnp.int32)
    for t in range(HB // 16):
        hist_v[pl.ds(t * 16, 16)] = zero16
    idp_v[pl.ds(0, 16)] = neg16
    idp_v[pl.ds(16, 16)] = neg16
    idp_v[pl.ds(PAD + N, 16)] = neg16
    idp_v[pl.ds(PAD + N + 16, 16)] = neg16

    def pass_a(j, _):
        idv = ids_v[pl.ds(j * 16, 16)]
        w = zero16
        e = zero16
        for s in range(1, 16):
            shf = _take16(idv, jnp.maximum(iota - s, 0))
            w = w + jnp.where((iota >= s) & (idv == shf), one16, zero16)
            shb = _take16(idv, jnp.minimum(iota + s, 15))
            e = e + jnp.where((iota < 16 - s) & (idv == shb), one16, zero16)
        cnt = plsc.load_gather(hist_v, [idv])
        r_v[pl.ds(j * 16, 16)] = cnt + w
        plsc.store_scatter(hist_v, [idv], cnt + w + e + 1, mask=(e == 0))
        return 0

    lax.fori_loop(0, VR, pass_a, 0)

    carry = jnp.int32(0)
    for t in range(HB // 16):
        h = hist_v[pl.ds(t * 16, 16)]
        inc = plsc.cumsum(h) + carry
        fx_v[pl.ds(t * 16, 16)] = inc - h
        carry = jnp.max(inc)

    def pass_b(j, _):
        idv = ids_v[pl.ds(j * 16, 16)]
        p = plsc.load_gather(fx_v, [idv]) + r_v[pl.ds(j * 16, 16)]
        pos_v[pl.ds(j * 16, 16)] = p + base
        plsc.store_scatter(sidx_v, [p], iota + (base + j * 16))
        plsc.store_scatter(idp_v, [p + PAD], idv)
        return 0

    lax.fori_loop(0, VR, pass_b, 0)

    pltpu.sync_copy(pos_v, pos_hbm.at[wid])
    pltpu.sync_copy(idp_v, idsrt_hbm.at[wid])
    for cc in range(NCHUNK):
        pltpu.async_copy(xf_hbm.at[sidx_v.at[pl.ds(cc * GCH, GCH)]],
                         gbuf, sem).wait()
        pltpu.sync_copy(gbuf, xs_hbm.at[pl.ds(base + cc * GCH, GCH)])


def _unsort_body(nslice, pos_hbm, os_hbm, out_hbm, pos_v, gbuf, sem):
    wid = lax.axis_index("s") * 2 + lax.axis_index("c")

    @pl.when(wid < nslice)
    def _():
        base = wid * N
        pltpu.sync_copy(pos_hbm.at[wid], pos_v)
        for cc in range(NCHUNK):
            pltpu.async_copy(os_hbm.at[pos_v.at[pl.ds(cc * GCH, GCH)]],
                             gbuf, sem).wait()
            pltpu.sync_copy(gbuf, out_hbm.at[pl.ds(base + cc * GCH, GCH)])


def _make_calls(BN):
    ids_call = pl.pallas_call(
        _ids_body,
        grid=(BN,),
        in_specs=[pl.BlockSpec((1, N, CP), lambda i: (i, 0, 0))],
        out_specs=pl.BlockSpec((1, 1, N), lambda i: (i, 0, 0)),
        out_shape=jax.ShapeDtypeStruct((BN, 1, N), jnp.int32),
    )

    wspec = pl.BlockSpec((CP, CP), lambda i: (0, 0))
    bspec = pl.BlockSpec((1, CP), lambda i: (0, 0))
    attn_call = pl.pallas_call(
        _attn_body,
        grid=(BN,),
        in_specs=[pl.BlockSpec((1, N, CP), lambda i: (i, 0, 0)),
                  pl.BlockSpec((1, N, 1), lambda i: (i, 0, 0)),
                  pl.BlockSpec((1, 1, NPAD), lambda i: (i, 0, 0)),
                  pl.BlockSpec((1, 1, NPAD), lambda i: (i, 0, 0)),
                  wspec, bspec, wspec, bspec, wspec, bspec, wspec, bspec,
                  wspec, bspec],
        out_specs=pl.BlockSpec((1, N, CP), lambda i: (i, 0, 0)),
        out_shape=jax.ShapeDtypeStruct((BN, N, CP), jnp.float32),
        scratch_shapes=[pltpu.VMEM((NPAD, CP), jnp.float32),
                        pltpu.VMEM((NPAD, CP), jnp.float32),
                        pltpu.VMEM((N, CP), jnp.float32),
                        pltpu.VMEM((N, CP), jnp.float32),
                        pltpu.VMEM((N, CP), jnp.float32)],
    )

    mesh = plsc.VectorSubcoreMesh(core_axis_name="c", subcore_axis_name="s")
    sort_call = functools.partial(
        pl.kernel,
        out_type=[jax.ShapeDtypeStruct((BN * N, CP), jnp.float32),
                  jax.ShapeDtypeStruct((BN, NPAD), jnp.int32),
                  jax.ShapeDtypeStruct((BN, N), jnp.int32)],
        mesh=mesh,
        compiler_params=pltpu.CompilerParams(needs_layout_passes=False),
        scratch_types=[pltpu.VMEM((N,), jnp.int32),
                       pltpu.VMEM((N,), jnp.int32),
                       pltpu.VMEM((N,), jnp.int32),
                       pltpu.VMEM((N,), jnp.int32),
                       pltpu.VMEM((NPAD,), jnp.int32),
                       pltpu.VMEM((HB,), jnp.int32),
                       pltpu.VMEM((HB,), jnp.int32),
                       pltpu.VMEM((GCH, CP), jnp.float32),
                       pltpu.SemaphoreType.DMA],
    )(functools.partial(_sort_gather_body, BN))

    unsort_call = functools.partial(
        pl.kernel,
        out_type=jax.ShapeDtypeStruct((BN * N, CP), jnp.float32),
        mesh=mesh,
        compiler_params=pltpu.CompilerParams(needs_layout_passes=False),
        scratch_types=[pltpu.VMEM((N,), jnp.int32),
                       pltpu.VMEM((GCH, CP), jnp.float32),
                       pltpu.SemaphoreType.DMA],
    )(functools.partial(_unsort_body, BN))

    return ids_call, sort_call, attn_call, unsort_call


def kernel(x, Wq, bq, Wk, bk, Wv, bv, Wp, bp, Wg, bg):
    B, Cc, H, W = x.shape
    gh, gw = H // (WS * GS), W // (WS * GS)
    BN = B * gh * gw

    ids_call, sort_call, attn_call, unsort_call = _make_calls(BN)
    wpad = lambda w: jnp.pad(w, ((0, CP - Cc), (0, CP - Cc)))
    bpad = lambda b: jnp.pad(b, (0, CP - Cc)).reshape(1, CP)
    wb = (wpad(Wq), bpad(bq), wpad(Wk), bpad(bk), wpad(Wv), bpad(bv),
          wpad(Wp), bpad(bp), wpad(Wg), bpad(bg))

    xg = x.reshape(B, Cc, gh, GS, WS, gw, GS, WS)
    xg = jnp.transpose(xg, (0, 2, 5, 3, 6, 4, 7, 1))
    xf = xg.reshape(BN, N, Cc)
    xf = jnp.pad(xf, ((0, 0), (0, 0), (0, CP - Cc)))

    ids3 = ids_call(xf)
    xs_flat, idsrt, pos = sort_call(ids3, xf.reshape(BN * N, CP))
    idc = idsrt[:, PAD:PAD + N].reshape(BN, N, 1)
    idr = idsrt.reshape(BN, 1, NPAD)
    idr2 = jnp.pad(idsrt[:, KW // 2:], ((0, 0), (0, KW // 2)),
                   constant_values=-1).reshape(BN, 1, NPAD)
    osort = attn_call(xs_flat.reshape(BN, N, CP), idc, idr, idr2, *wb)
    ofin = unsort_call(pos, osort.reshape(BN * N, CP))

    out = ofin.reshape(B, gh, gw, GS, GS, WS, WS, CP)
    out = jnp.transpose(out, (0, 7, 1, 3, 5, 2, 4, 6))
    return out[:, :Cc].reshape(B, Cc, H, W)
